# baseline (speedup 1.0000x reference)
import functools

import jax
import jax.numpy as jnp
from jax import lax
from jax.experimental import pallas as pl
from jax.experimental.pallas import tpu as pltpu
from jax.experimental.pallas import tpu_sc as plsc

N = 100000
NPAD = 100096
E = 3200000
RL = 128
EROWS = 25728
EPAD = EROWS * RL
K = 6
NSUB = 16
NPS = NPAD // NSUB
RB = 2000
EPS = 1e-5


def _sc_segsum(tables, edges, zeros):
    chunks_total = EROWS // K
    chunks_per_tile = chunks_total // NSUB
    pairs = chunks_per_tile // 2

    mesh = plsc.VectorSubcoreMesh(core_axis_name="c", subcore_axis_name="s")

    @functools.partial(
        pl.kernel,
        mesh=mesh,
        out_type=jax.ShapeDtypeStruct((2, NPAD, 16), jnp.float32),
        scratch_types=[
            pltpu.VMEM((2, K, RL), jnp.int32),
            pltpu.VMEM((2, K, RL), jnp.int32),
            pltpu.VMEM((2, K, RL, 16), jnp.float32),
            pltpu.VMEM_SHARED((NPAD, 16), jnp.float32),
            pltpu.SemaphoreType.DMA,
            pltpu.SemaphoreType.DMA,
            pltpu.SemaphoreType.DMA,
            pltpu.SemaphoreType.DMA,
        ],
        compiler_params=pltpu.CompilerParams(use_tc_tiling_on_sc=False),
    )
    def segsum(tables_hbm, edges_hbm, zeros_hbm, out_hbm,
               sidx, didx, rows, acc, gsem0, gsem1, ssem0, ssem1):
        cid = lax.axis_index("c")
        sid = lax.axis_index("s")
        gsem = (gsem0, gsem1)
        ssem = (ssem0, ssem1)
        pltpu.sync_copy(zeros_hbm.at[pl.ds(sid * NPS, NPS)],
                        acc.at[pl.ds(sid * NPS, NPS)])
        plsc.subcore_barrier()

        base = sid * chunks_per_tile

        def load_and_gather(cb, b):
            pltpu.sync_copy(edges_hbm.at[0].at[pl.ds(cb * K, K)], sidx.at[b])
            pltpu.sync_copy(edges_hbm.at[1].at[pl.ds(cb * K, K)], didx.at[b])
            for j in range(K):
                pltpu.async_copy(tables_hbm.at[cid].at[sidx.at[b].at[j]],
                                 rows.at[b].at[j], gsem[b])

        def wait_gathers(b):
            for j in range(K):
                pltpu.make_async_copy(tables_hbm.at[cid].at[sidx.at[b].at[j]],
                                      rows.at[b].at[j], gsem[b]).wait()

        def scatter_and_drain(b):
            cps = [
                pltpu.async_copy(rows.at[b].at[j], acc.at[didx.at[b].at[j]],
                                 ssem[b], add=True)
                for j in range(K)
            ]
            for cp in cps:
                cp.wait()

        for b in range(2):
            load_and_gather(base + b, b)

        def pair(o, carry):
            for b in range(2):
                cb = base + o * 2 + b
                wait_gathers(b)
                scatter_and_drain(b)

                @pl.when(o < pairs - 1)
                def _():
                    load_and_gather(cb + 2, b)
            return carry

        lax.fori_loop(0, pairs, pair, 0)
        plsc.subcore_barrier()
        pltpu.sync_copy(acc.at[pl.ds(sid * NPS, NPS)],
                        out_hbm.at[cid].at[pl.ds(sid * NPS, NPS)])

    return segsum(tables, edges, zeros)


def _tc_a_body(sums_ref, x_ref, wl_ref, bl_ref, wr_ref,
               h1pre_ref, stats_ref, ssum, ssq):
    i = pl.program_id(0)
    s = sums_ref[...]
    agg = jnp.concatenate([s[0], s[1][:, :4]], axis=1)
    cnt = jnp.maximum(s[1][:, 4:5], 1.0)
    agg = agg / cnt
    z = (lax.dot_general(agg, wl_ref[...], (((1,), (1,)), ((), ())),
                         preferred_element_type=jnp.float32)
         + bl_ref[...]
         + lax.dot_general(x_ref[...], wr_ref[...], (((1,), (1,)), ((), ())),
                           preferred_element_type=jnp.float32))
    h1pre_ref[...] = z

    @pl.when(i == 0)
    def _():
        ssum[...] = jnp.zeros_like(ssum)
        ssq[...] = jnp.zeros_like(ssq)

    ssum[...] += jnp.sum(z, axis=0, keepdims=True)
    ssq[...] += jnp.sum(z * z, axis=0, keepdims=True)

    @pl.when(i == pl.num_programs(0) - 1)
    def _():
        stats_ref[...] = jnp.concatenate([ssum[...], ssq[...]], axis=0)


def _tc_a(sums1, x, Wl1, bl1, Wr1):
    grid = (N // RB,)
    return pl.pallas_call(
        _tc_a_body,
        grid=grid,
        in_specs=[
            pl.BlockSpec((2, RB, 16), lambda i: (0, i, 0)),
            pl.BlockSpec((RB, 20), lambda i: (i, 0)),
            pl.BlockSpec((64, 20), lambda i: (0, 0)),
            pl.BlockSpec((1, 64), lambda i: (0, 0)),
            pl.BlockSpec((64, 20), lambda i: (0, 0)),
        ],
        out_specs=[
            pl.BlockSpec((RB, 64), lambda i: (i, 0)),
            pl.BlockSpec((2, 64), lambda i: (0, 0)),
        ],
        out_shape=[
            jax.ShapeDtypeStruct((N, 64), jnp.float32),
            jax.ShapeDtypeStruct((2, 64), jnp.float32),
        ],
        scratch_shapes=[
            pltpu.VMEM((1, 64), jnp.float32),
            pltpu.VMEM((1, 64), jnp.float32),
        ],
    )(sums1, x, Wl1, bl1, Wr1)


def _tc_b_body(h1pre_ref, stats_ref, g_ref, be_ref, wl2_ref, bl2_ref, wr2_ref,
               p2_ref, r2_ref):
    st = stats_ref[...]
    mu = st[0:1, :] / N
    var = st[1:2, :] / N - mu * mu
    inv = lax.rsqrt(var + EPS)
    z = h1pre_ref[...]
    h1 = jnp.maximum((z - mu) * inv * g_ref[...] + be_ref[...], 0.0)
    p2 = lax.dot_general(h1, wl2_ref[...], (((1,), (1,)), ((), ())),
                         preferred_element_type=jnp.float32)
    p2_ref[0] = p2[:, :16]
    p2_ref[1] = p2[:, 16:]
    r2_ref[...] = (lax.dot_general(h1, wr2_ref[...], (((1,), (1,)), ((), ())),
                                   preferred_element_type=jnp.float32)
                   + bl2_ref[...])


def _tc_b(h1pre, stats1, g1, be1, Wl2, bl2, Wr2):
    grid = (N // RB,)
    return pl.pallas_call(
        _tc_b_body,
        grid=grid,
        in_specs=[
            pl.BlockSpec((RB, 64), lambda i: (i, 0)),
            pl.BlockSpec((2, 64), lambda i: (0, 0)),
            pl.BlockSpec((1, 64), lambda i: (0, 0)),
            pl.BlockSpec((1, 64), lambda i: (0, 0)),
            pl.BlockSpec((32, 64), lambda i: (0, 0)),
            pl.BlockSpec((1, 32), lambda i: (0, 0)),
            pl.BlockSpec((32, 64), lambda i: (0, 0)),
        ],
        out_specs=[
            pl.BlockSpec((2, RB, 16), lambda i: (0, i, 0)),
            pl.BlockSpec((RB, 32), lambda i: (i, 0)),
        ],
        out_shape=[
            jax.ShapeDtypeStruct((2, NPAD, 16), jnp.float32),
            jax.ShapeDtypeStruct((N, 32), jnp.float32),
        ],
    )(h1pre, stats1, g1, be1, Wl2, bl2, Wr2)


def _tc_c_body(sums_ref, cnt_ref, r2_ref, h2pre_ref, stats_ref, ssum, ssq):
    i = pl.program_id(0)
    s = sums_ref[...]
    agg = jnp.concatenate([s[0], s[1]], axis=1)
    cnt = jnp.maximum(cnt_ref[...], 1.0)
    z = agg / cnt + r2_ref[...]
    h2pre_ref[...] = z

    @pl.when(i == 0)
    def _():
        ssum[...] = jnp.zeros_like(ssum)
        ssq[...] = jnp.zeros_like(ssq)

    ssum[...] += jnp.sum(z, axis=0, keepdims=True)
    ssq[...] += jnp.sum(z * z, axis=0, keepdims=True)

    @pl.when(i == pl.num_programs(0) - 1)
    def _():
        stats_ref[...] = jnp.concatenate([ssum[...], ssq[...]], axis=0)


def _tc_c(sums2, cnt, r2):
    grid = (N // RB,)
    return pl.pallas_call(
        _tc_c_body,
        grid=grid,
        in_specs=[
            pl.BlockSpec((2, RB, 16), lambda i: (0, i, 0)),
            pl.BlockSpec((RB, 1), lambda i: (i, 0)),
            pl.BlockSpec((RB, 32), lambda i: (i, 0)),
        ],
        out_specs=[
            pl.BlockSpec((RB, 32), lambda i: (i, 0)),
            pl.BlockSpec((2, 32), lambda i: (0, 0)),
        ],
        out_shape=[
            jax.ShapeDtypeStruct((N, 32), jnp.float32),
            jax.ShapeDtypeStruct((2, 32), jnp.float32),
        ],
        scratch_shapes=[
            pltpu.VMEM((1, 32), jnp.float32),
            pltpu.VMEM((1, 32), jnp.float32),
        ],
    )(sums2, cnt, r2)


def _tc_d_body(h2pre_ref, stats_ref, g_ref, be_ref, wh1_ref, bh1_ref,
               wh2_ref, bh2_ref, out_ref):
    st = stats_ref[...]
    mu = st[0:1, :] / N
    var = st[1:2, :] / N - mu * mu
    inv = lax.rsqrt(var + EPS)
    z = h2pre_ref[...]
    h2 = jnp.maximum((z - mu) * inv * g_ref[...] + be_ref[...], 0.0)
    h3 = jnp.maximum(
        lax.dot_general(h2, wh1_ref[...], (((1,), (1,)), ((), ())),
                        preferred_element_type=jnp.float32) + bh1_ref[...],
        0.0)
    out_ref[...] = (lax.dot_general(h3, wh2_ref[...], (((1,), (1,)), ((), ())),
                                    preferred_element_type=jnp.float32)
                    + bh2_ref[0, 0])


def _tc_d(h2pre, stats2, g2, be2, Wh1, bh1, Wh2, bh2):
    grid = (N // RB,)
    return pl.pallas_call(
        _tc_d_body,
        grid=grid,
        in_specs=[
            pl.BlockSpec((RB, 32), lambda i: (i, 0)),
            pl.BlockSpec((2, 32), lambda i: (0, 0)),
            pl.BlockSpec((1, 32), lambda i: (0, 0)),
            pl.BlockSpec((1, 32), lambda i: (0, 0)),
            pl.BlockSpec((16, 32), lambda i: (0, 0)),
            pl.BlockSpec((1, 16), lambda i: (0, 0)),
            pl.BlockSpec((8, 16), lambda i: (0, 0)),
            pl.BlockSpec(memory_space=pltpu.SMEM),
        ],
        out_specs=pl.BlockSpec((RB, 8), lambda i: (i, 0)),
        out_shape=jax.ShapeDtypeStruct((N, 8), jnp.float32),
    )(h2pre, stats2, g2, be2, Wh1, bh1, Wh2, bh2)


def kernel(xs, xt, edge_index, Wl1, bl1, Wr1, g1, be1, Wl2, bl2, Wr2, g2, be2,
           Wh1, bh1, Wh2, bh2):
    x = jnp.concatenate([xs, xt], axis=-1)
    edges = jnp.pad(edge_index.astype(jnp.int32), ((0, 0), (0, EPAD - E)),
                    constant_values=NPAD - 1).reshape(2, EROWS, RL)
    zeros = jnp.zeros((NPAD, 16), jnp.float32)

    t1b = jnp.concatenate(
        [x[:, 16:20], jnp.ones((N, 1), jnp.float32),
         jnp.zeros((N, 11), jnp.float32)], axis=1)
    tables1 = jnp.pad(jnp.stack([x[:, :16], t1b]),
                      ((0, 0), (0, NPAD - N), (0, 0)))

    sums1 = _sc_segsum(tables1, edges, zeros)
    cnt = lax.slice(sums1[1], (0, 4), (N, 5))

    h1pre, stats1 = _tc_a(sums1, x, Wl1, bl1.reshape(1, 64), Wr1)
    p2, r2 = _tc_b(h1pre, stats1, g1.reshape(1, 64), be1.reshape(1, 64),
                   Wl2, bl2.reshape(1, 32), Wr2)

    sums2 = _sc_segsum(p2, edges, zeros)

    h2pre, stats2 = _tc_c(sums2, cnt, r2)
    out = _tc_d(h2pre, stats2, g2.reshape(1, 32), be2.reshape(1, 32),
                Wh1, bh1.reshape(1, 16),
                jnp.pad(Wh2, ((0, 7), (0, 0))), bh2.reshape(1, 1))
    return out[:, 0]


# speedup vs baseline: 1.5349x; 1.5349x over previous
import functools

import jax
import jax.numpy as jnp
from jax import lax
from jax.experimental import pallas as pl
from jax.experimental.pallas import tpu as pltpu
from jax.experimental.pallas import tpu_sc as plsc

N = 100000
NPAD = 102400
E = 3200000
RL = 128
EROWS = 25728
EPAD = EROWS * RL
K = 6
NSUB = 16
NPS = NPAD // NSUB
RB = 2000
EPS = 1e-5


def _sc_segsum(tables, edges, zeros):
    chunks_total = EROWS // K
    chunks_per_tile = chunks_total // NSUB
    pairs = chunks_per_tile // 2

    mesh = plsc.VectorSubcoreMesh(core_axis_name="c", subcore_axis_name="s")

    @functools.partial(
        pl.kernel,
        mesh=mesh,
        out_type=jax.ShapeDtypeStruct((2, NPAD, 16), jnp.float32),
        scratch_types=[
            pltpu.VMEM((2, K, RL), jnp.int32),
            pltpu.VMEM((2, K, RL), jnp.int32),
            pltpu.VMEM((2, K, RL, 16), jnp.float32),
            pltpu.VMEM_SHARED((NPAD, 16), jnp.float32),
            pltpu.SemaphoreType.DMA,
            pltpu.SemaphoreType.DMA,
            pltpu.SemaphoreType.DMA,
            pltpu.SemaphoreType.DMA,
        ],
        compiler_params=pltpu.CompilerParams(use_tc_tiling_on_sc=False),
    )
    def segsum(tables_hbm, edges_hbm, zeros_hbm, out_hbm,
               sidx, didx, rows, acc, gsem0, gsem1, ssem0, ssem1):
        cid = lax.axis_index("c")
        sid = lax.axis_index("s")
        gsem = (gsem0, gsem1)
        ssem = (ssem0, ssem1)
        pltpu.sync_copy(zeros_hbm.at[pl.ds(sid * NPS, NPS)],
                        acc.at[pl.ds(sid * NPS, NPS)])
        plsc.subcore_barrier()

        base = sid * chunks_per_tile

        def load_and_gather(cb, b):
            pltpu.sync_copy(edges_hbm.at[0].at[pl.ds(cb * K, K)], sidx.at[b])
            pltpu.sync_copy(edges_hbm.at[1].at[pl.ds(cb * K, K)], didx.at[b])
            for j in range(K):
                pltpu.async_copy(tables_hbm.at[cid].at[sidx.at[b].at[j]],
                                 rows.at[b].at[j], gsem[b])

        def wait_gathers(b):
            for j in range(K):
                pltpu.make_async_copy(tables_hbm.at[cid].at[sidx.at[b].at[j]],
                                      rows.at[b].at[j], gsem[b]).wait()

        def scatter_and_drain(b):
            cps = [
                pltpu.async_copy(rows.at[b].at[j], acc.at[didx.at[b].at[j]],
                                 ssem[b], add=True)
                for j in range(K)
            ]
            for cp in cps:
                cp.wait()

        for b in range(2):
            load_and_gather(base + b, b)

        def pair(o, carry):
            for b in range(2):
                cb = base + o * 2 + b
                wait_gathers(b)
                scatter_and_drain(b)

                @pl.when(o < pairs - 1)
                def _():
                    load_and_gather(cb + 2, b)
            return carry

        lax.fori_loop(0, pairs, pair, 0)
        plsc.subcore_barrier()
        pltpu.sync_copy(acc.at[pl.ds(sid * NPS, NPS)],
                        out_hbm.at[cid].at[pl.ds(sid * NPS, NPS)])

    return segsum(tables, edges, zeros)


def _tc_a_body(sums_ref, x_ref, wl_ref, bl_ref, wr_ref,
               h1pre_ref, stats_ref, ssum, ssq):
    i = pl.program_id(0)
    s = sums_ref[...]
    agg = jnp.concatenate([s[0], s[1][:, :4]], axis=1)
    cnt = jnp.maximum(s[1][:, 4:5], 1.0)
    agg = agg / cnt
    z = (lax.dot_general(agg, wl_ref[...], (((1,), (1,)), ((), ())),
                         preferred_element_type=jnp.float32)
         + bl_ref[...]
         + lax.dot_general(x_ref[...], wr_ref[...], (((1,), (1,)), ((), ())),
                           preferred_element_type=jnp.float32))
    h1pre_ref[...] = z

    @pl.when(i == 0)
    def _():
        ssum[...] = jnp.zeros_like(ssum)
        ssq[...] = jnp.zeros_like(ssq)

    ssum[...] += jnp.sum(z, axis=0, keepdims=True)
    ssq[...] += jnp.sum(z * z, axis=0, keepdims=True)

    @pl.when(i == pl.num_programs(0) - 1)
    def _():
        stats_ref[...] = jnp.concatenate([ssum[...], ssq[...]], axis=0)


def _tc_a(sums1, x, Wl1, bl1, Wr1):
    grid = (N // RB,)
    return pl.pallas_call(
        _tc_a_body,
        grid=grid,
        in_specs=[
            pl.BlockSpec((2, RB, 16), lambda i: (0, i, 0)),
            pl.BlockSpec((RB, 20), lambda i: (i, 0)),
            pl.BlockSpec((64, 20), lambda i: (0, 0)),
            pl.BlockSpec((1, 64), lambda i: (0, 0)),
            pl.BlockSpec((64, 20), lambda i: (0, 0)),
        ],
        out_specs=[
            pl.BlockSpec((RB, 64), lambda i: (i, 0)),
            pl.BlockSpec((2, 64), lambda i: (0, 0)),
        ],
        out_shape=[
            jax.ShapeDtypeStruct((N, 64), jnp.float32),
            jax.ShapeDtypeStruct((2, 64), jnp.float32),
        ],
        scratch_shapes=[
            pltpu.VMEM((1, 64), jnp.float32),
            pltpu.VMEM((1, 64), jnp.float32),
        ],
    )(sums1, x, Wl1, bl1, Wr1)


def _tc_b_body(h1pre_ref, stats_ref, g_ref, be_ref, wl2_ref, bl2_ref, wr2_ref,
               p2_ref, r2_ref):
    st = stats_ref[...]
    mu = st[0:1, :] / N
    var = st[1:2, :] / N - mu * mu
    inv = lax.rsqrt(var + EPS)
    z = h1pre_ref[...]
    h1 = jnp.maximum((z - mu) * inv * g_ref[...] + be_ref[...], 0.0)
    p2 = lax.dot_general(h1, wl2_ref[...], (((1,), (1,)), ((), ())),
                         preferred_element_type=jnp.float32)
    p2_ref[0] = p2[:, :16]
    p2_ref[1] = p2[:, 16:]
    r2_ref[...] = (lax.dot_general(h1, wr2_ref[...], (((1,), (1,)), ((), ())),
                                   preferred_element_type=jnp.float32)
                   + bl2_ref[...])


def _tc_b(h1pre, stats1, g1, be1, Wl2, bl2, Wr2):
    grid = (N // RB,)
    return pl.pallas_call(
        _tc_b_body,
        grid=grid,
        in_specs=[
            pl.BlockSpec((RB, 64), lambda i: (i, 0)),
            pl.BlockSpec((2, 64), lambda i: (0, 0)),
            pl.BlockSpec((1, 64), lambda i: (0, 0)),
            pl.BlockSpec((1, 64), lambda i: (0, 0)),
            pl.BlockSpec((32, 64), lambda i: (0, 0)),
            pl.BlockSpec((1, 32), lambda i: (0, 0)),
            pl.BlockSpec((32, 64), lambda i: (0, 0)),
        ],
        out_specs=[
            pl.BlockSpec((2, RB, 16), lambda i: (0, i, 0)),
            pl.BlockSpec((RB, 32), lambda i: (i, 0)),
        ],
        out_shape=[
            jax.ShapeDtypeStruct((2, NPAD, 16), jnp.float32),
            jax.ShapeDtypeStruct((N, 32), jnp.float32),
        ],
    )(h1pre, stats1, g1, be1, Wl2, bl2, Wr2)


def _tc_c_body(sums_ref, cnt_ref, r2_ref, h2pre_ref, stats_ref, ssum, ssq):
    i = pl.program_id(0)
    s = sums_ref[...]
    agg = jnp.concatenate([s[0], s[1]], axis=1)
    cnt = jnp.maximum(cnt_ref[...], 1.0)
    z = agg / cnt + r2_ref[...]
    h2pre_ref[...] = z

    @pl.when(i == 0)
    def _():
        ssum[...] = jnp.zeros_like(ssum)
        ssq[...] = jnp.zeros_like(ssq)

    ssum[...] += jnp.sum(z, axis=0, keepdims=True)
    ssq[...] += jnp.sum(z * z, axis=0, keepdims=True)

    @pl.when(i == pl.num_programs(0) - 1)
    def _():
        stats_ref[...] = jnp.concatenate([ssum[...], ssq[...]], axis=0)


def _tc_c(sums2, cnt, r2):
    grid = (N // RB,)
    return pl.pallas_call(
        _tc_c_body,
        grid=grid,
        in_specs=[
            pl.BlockSpec((2, RB, 16), lambda i: (0, i, 0)),
            pl.BlockSpec((RB, 1), lambda i: (i, 0)),
            pl.BlockSpec((RB, 32), lambda i: (i, 0)),
        ],
        out_specs=[
            pl.BlockSpec((RB, 32), lambda i: (i, 0)),
            pl.BlockSpec((2, 32), lambda i: (0, 0)),
        ],
        out_shape=[
            jax.ShapeDtypeStruct((N, 32), jnp.float32),
            jax.ShapeDtypeStruct((2, 32), jnp.float32),
        ],
        scratch_shapes=[
            pltpu.VMEM((1, 32), jnp.float32),
            pltpu.VMEM((1, 32), jnp.float32),
        ],
    )(sums2, cnt, r2)


def _tc_d_body(h2pre_ref, stats_ref, g_ref, be_ref, wh1_ref, bh1_ref,
               wh2_ref, bh2_ref, out_ref):
    st = stats_ref[...]
    mu = st[0:1, :] / N
    var = st[1:2, :] / N - mu * mu
    inv = lax.rsqrt(var + EPS)
    z = h2pre_ref[...]
    h2 = jnp.maximum((z - mu) * inv * g_ref[...] + be_ref[...], 0.0)
    h3 = jnp.maximum(
        lax.dot_general(h2, wh1_ref[...], (((1,), (1,)), ((), ())),
                        preferred_element_type=jnp.float32) + bh1_ref[...],
        0.0)
    out_ref[...] = (lax.dot_general(h3, wh2_ref[...], (((1,), (1,)), ((), ())),
                                    preferred_element_type=jnp.float32)
                    + bh2_ref[0, 0])


def _tc_d(h2pre, stats2, g2, be2, Wh1, bh1, Wh2, bh2):
    grid = (N // RB,)
    return pl.pallas_call(
        _tc_d_body,
        grid=grid,
        in_specs=[
            pl.BlockSpec((RB, 32), lambda i: (i, 0)),
            pl.BlockSpec((2, 32), lambda i: (0, 0)),
            pl.BlockSpec((1, 32), lambda i: (0, 0)),
            pl.BlockSpec((1, 32), lambda i: (0, 0)),
            pl.BlockSpec((16, 32), lambda i: (0, 0)),
            pl.BlockSpec((1, 16), lambda i: (0, 0)),
            pl.BlockSpec((8, 16), lambda i: (0, 0)),
            pl.BlockSpec(memory_space=pltpu.SMEM),
        ],
        out_specs=pl.BlockSpec((RB, 8), lambda i: (i, 0)),
        out_shape=jax.ShapeDtypeStruct((N, 8), jnp.float32),
    )(h2pre, stats2, g2, be2, Wh1, bh1, Wh2, bh2)


def kernel(xs, xt, edge_index, Wl1, bl1, Wr1, g1, be1, Wl2, bl2, Wr2, g2, be2,
           Wh1, bh1, Wh2, bh2):
    x = jnp.concatenate([xs, xt], axis=-1)
    pad_idx = N + jnp.arange(EPAD - E, dtype=jnp.int32) % (NPAD - N)
    edges = jnp.concatenate(
        [edge_index.astype(jnp.int32),
         jnp.broadcast_to(pad_idx, (2, EPAD - E))], axis=1
    ).reshape(2, EROWS, RL)
    zeros = jnp.zeros((NPAD, 16), jnp.float32)

    t1b = jnp.concatenate(
        [x[:, 16:20], jnp.ones((N, 1), jnp.float32),
         jnp.zeros((N, 11), jnp.float32)], axis=1)
    tables1 = jnp.pad(jnp.stack([x[:, :16], t1b]),
                      ((0, 0), (0, NPAD - N), (0, 0)))

    sums1 = _sc_segsum(tables1, edges, zeros)
    cnt = lax.slice(sums1[1], (0, 4), (N, 5))

    h1pre, stats1 = _tc_a(sums1, x, Wl1, bl1.reshape(1, 64), Wr1)
    p2, r2 = _tc_b(h1pre, stats1, g1.reshape(1, 64), be1.reshape(1, 64),
                   Wl2, bl2.reshape(1, 32), Wr2)

    sums2 = _sc_segsum(p2, edges, zeros)

    h2pre, stats2 = _tc_c(sums2, cnt, r2)
    out = _tc_d(h2pre, stats2, g2.reshape(1, 32), be2.reshape(1, 32),
                Wh1, bh1.reshape(1, 16),
                jnp.pad(Wh2, ((0, 7), (0, 0))), bh2.reshape(1, 1))
    return out[:, 0]


# speedup vs baseline: 1.5511x; 1.0106x over previous
import functools

import jax
import jax.numpy as jnp
from jax import lax
from jax.experimental import pallas as pl
from jax.experimental.pallas import tpu as pltpu
from jax.experimental.pallas import tpu_sc as plsc

N = 100000
NPAD = 102400
E = 3200000
RL = 128
EROWS = 25728
EPAD = EROWS * RL
K = 6
NSUB = 16
NPS = NPAD // NSUB
RB = 2000
EPS = 1e-5


def _sc_segsum(tables, edges, zeros):
    chunks_total = EROWS // K
    chunks_per_tile = chunks_total // NSUB
    pairs = chunks_per_tile // 2

    mesh = plsc.VectorSubcoreMesh(core_axis_name="c", subcore_axis_name="s")

    @functools.partial(
        pl.kernel,
        mesh=mesh,
        out_type=jax.ShapeDtypeStruct((2, NPAD, 16), jnp.float32),
        scratch_types=[
            pltpu.VMEM((2, K, RL), jnp.int32),
            pltpu.VMEM((2, K, RL), jnp.int32),
            pltpu.VMEM((2, K, RL, 16), jnp.float32),
            pltpu.VMEM_SHARED((NPAD, 16), jnp.float32),
            pltpu.SemaphoreType.DMA,
            pltpu.SemaphoreType.DMA,
            pltpu.SemaphoreType.DMA,
            pltpu.SemaphoreType.DMA,
        ],
        compiler_params=pltpu.CompilerParams(use_tc_tiling_on_sc=False),
    )
    def segsum(tables_hbm, edges_hbm, zeros_hbm, out_hbm,
               sidx, didx, rows, acc, gsem0, gsem1, ssem0, ssem1):
        cid = lax.axis_index("c")
        sid = lax.axis_index("s")
        gsem = (gsem0, gsem1)
        ssem = (ssem0, ssem1)
        pltpu.sync_copy(zeros_hbm.at[pl.ds(sid * NPS, NPS)],
                        acc.at[pl.ds(sid * NPS, NPS)])
        plsc.subcore_barrier()

        base = sid * chunks_per_tile

        HL = RL // 2

        def load_and_gather(cb, b):
            pltpu.sync_copy(edges_hbm.at[0].at[pl.ds(cb * K, K)], sidx.at[b])
            pltpu.sync_copy(edges_hbm.at[1].at[pl.ds(cb * K, K)], didx.at[b])
            for j in range(K):
                for h in range(2):
                    pltpu.async_copy(
                        tables_hbm.at[cid].at[sidx.at[b].at[j, pl.ds(h * HL, HL)]],
                        rows.at[b].at[j].at[pl.ds(h * HL, HL)], gsem[b])

        def wait_gathers(b):
            for j in range(K):
                for h in range(2):
                    pltpu.make_async_copy(
                        tables_hbm.at[cid].at[sidx.at[b].at[j, pl.ds(h * HL, HL)]],
                        rows.at[b].at[j].at[pl.ds(h * HL, HL)], gsem[b]).wait()

        def scatter_and_drain(b):
            cps = [
                pltpu.async_copy(rows.at[b].at[j], acc.at[didx.at[b].at[j]],
                                 ssem[b], add=True)
                for j in range(K)
            ]
            for cp in cps:
                cp.wait()

        for b in range(2):
            load_and_gather(base + b, b)

        def pair(o, carry):
            for b in range(2):
                cb = base + o * 2 + b
                wait_gathers(b)
                scatter_and_drain(b)

                @pl.when(o < pairs - 1)
                def _():
                    load_and_gather(cb + 2, b)
            return carry

        lax.fori_loop(0, pairs, pair, 0)
        plsc.subcore_barrier()
        pltpu.sync_copy(acc.at[pl.ds(sid * NPS, NPS)],
                        out_hbm.at[cid].at[pl.ds(sid * NPS, NPS)])

    return segsum(tables, edges, zeros)


def _tc_a_body(sums_ref, x_ref, wl_ref, bl_ref, wr_ref,
               h1pre_ref, stats_ref, cnt8_ref, ssum, ssq):
    i = pl.program_id(0)
    s = sums_ref[...]
    cnt8_ref[...] = s[1][:, :8]
    agg = jnp.concatenate([s[0], s[1][:, :4]], axis=1)
    cnt = jnp.maximum(s[1][:, 4:5], 1.0)
    agg = agg / cnt
    z = (lax.dot_general(agg, wl_ref[...], (((1,), (1,)), ((), ())),
                         preferred_element_type=jnp.float32)
         + bl_ref[...]
         + lax.dot_general(x_ref[...], wr_ref[...], (((1,), (1,)), ((), ())),
                           preferred_element_type=jnp.float32))
    h1pre_ref[...] = z

    @pl.when(i == 0)
    def _():
        ssum[...] = jnp.zeros_like(ssum)
        ssq[...] = jnp.zeros_like(ssq)

    ssum[...] += jnp.sum(z, axis=0, keepdims=True)
    ssq[...] += jnp.sum(z * z, axis=0, keepdims=True)

    @pl.when(i == pl.num_programs(0) - 1)
    def _():
        stats_ref[...] = jnp.concatenate([ssum[...], ssq[...]], axis=0)


def _tc_a(sums1, x, Wl1, bl1, Wr1):
    grid = (N // RB,)
    return pl.pallas_call(
        _tc_a_body,
        grid=grid,
        in_specs=[
            pl.BlockSpec((2, RB, 16), lambda i: (0, i, 0)),
            pl.BlockSpec((RB, 20), lambda i: (i, 0)),
            pl.BlockSpec((64, 20), lambda i: (0, 0)),
            pl.BlockSpec((1, 64), lambda i: (0, 0)),
            pl.BlockSpec((64, 20), lambda i: (0, 0)),
        ],
        out_specs=[
            pl.BlockSpec((RB, 64), lambda i: (i, 0)),
            pl.BlockSpec((2, 64), lambda i: (0, 0)),
            pl.BlockSpec((RB, 8), lambda i: (i, 0)),
        ],
        out_shape=[
            jax.ShapeDtypeStruct((N, 64), jnp.float32),
            jax.ShapeDtypeStruct((2, 64), jnp.float32),
            jax.ShapeDtypeStruct((N, 8), jnp.float32),
        ],
        scratch_shapes=[
            pltpu.VMEM((1, 64), jnp.float32),
            pltpu.VMEM((1, 64), jnp.float32),
        ],
    )(sums1, x, Wl1, bl1, Wr1)


def _tc_b_body(h1pre_ref, stats_ref, g_ref, be_ref, wl2_ref, bl2_ref, wr2_ref,
               p2_ref, r2_ref):
    st = stats_ref[...]
    mu = st[0:1, :] / N
    var = st[1:2, :] / N - mu * mu
    inv = lax.rsqrt(var + EPS)
    z = h1pre_ref[...]
    h1 = jnp.maximum((z - mu) * inv * g_ref[...] + be_ref[...], 0.0)
    p2 = lax.dot_general(h1, wl2_ref[...], (((1,), (1,)), ((), ())),
                         preferred_element_type=jnp.float32)
    p2_ref[0] = p2[:, :16]
    p2_ref[1] = p2[:, 16:]
    r2_ref[...] = (lax.dot_general(h1, wr2_ref[...], (((1,), (1,)), ((), ())),
                                   preferred_element_type=jnp.float32)
                   + bl2_ref[...])


def _tc_b(h1pre, stats1, g1, be1, Wl2, bl2, Wr2):
    grid = (N // RB,)
    return pl.pallas_call(
        _tc_b_body,
        grid=grid,
        in_specs=[
            pl.BlockSpec((RB, 64), lambda i: (i, 0)),
            pl.BlockSpec((2, 64), lambda i: (0, 0)),
            pl.BlockSpec((1, 64), lambda i: (0, 0)),
            pl.BlockSpec((1, 64), lambda i: (0, 0)),
            pl.BlockSpec((32, 64), lambda i: (0, 0)),
            pl.BlockSpec((1, 32), lambda i: (0, 0)),
            pl.BlockSpec((32, 64), lambda i: (0, 0)),
        ],
        out_specs=[
            pl.BlockSpec((2, RB, 16), lambda i: (0, i, 0)),
            pl.BlockSpec((RB, 32), lambda i: (i, 0)),
        ],
        out_shape=[
            jax.ShapeDtypeStruct((2, NPAD, 16), jnp.float32),
            jax.ShapeDtypeStruct((N, 32), jnp.float32),
        ],
    )(h1pre, stats1, g1, be1, Wl2, bl2, Wr2)


def _tc_c_body(sums_ref, cnt_ref, r2_ref, h2pre_ref, stats_ref, ssum, ssq):
    i = pl.program_id(0)
    s = sums_ref[...]
    agg = jnp.concatenate([s[0], s[1]], axis=1)
    cnt = jnp.maximum(cnt_ref[:, 4:5], 1.0)
    z = agg / cnt + r2_ref[...]
    h2pre_ref[...] = z

    @pl.when(i == 0)
    def _():
        ssum[...] = jnp.zeros_like(ssum)
        ssq[...] = jnp.zeros_like(ssq)

    ssum[...] += jnp.sum(z, axis=0, keepdims=True)
    ssq[...] += jnp.sum(z * z, axis=0, keepdims=True)

    @pl.when(i == pl.num_programs(0) - 1)
    def _():
        stats_ref[...] = jnp.concatenate([ssum[...], ssq[...]], axis=0)


def _tc_c(sums2, cnt, r2):
    grid = (N // RB,)
    return pl.pallas_call(
        _tc_c_body,
        grid=grid,
        in_specs=[
            pl.BlockSpec((2, RB, 16), lambda i: (0, i, 0)),
            pl.BlockSpec((RB, 8), lambda i: (i, 0)),
            pl.BlockSpec((RB, 32), lambda i: (i, 0)),
        ],
        out_specs=[
            pl.BlockSpec((RB, 32), lambda i: (i, 0)),
            pl.BlockSpec((2, 32), lambda i: (0, 0)),
        ],
        out_shape=[
            jax.ShapeDtypeStruct((N, 32), jnp.float32),
            jax.ShapeDtypeStruct((2, 32), jnp.float32),
        ],
        scratch_shapes=[
            pltpu.VMEM((1, 32), jnp.float32),
            pltpu.VMEM((1, 32), jnp.float32),
        ],
    )(sums2, cnt, r2)


def _tc_d_body(h2pre_ref, stats_ref, g_ref, be_ref, wh1_ref, bh1_ref,
               wh2_ref, bh2_ref, out_ref):
    st = stats_ref[...]
    mu = st[0:1, :] / N
    var = st[1:2, :] / N - mu * mu
    inv = lax.rsqrt(var + EPS)
    z = h2pre_ref[...]
    h2 = jnp.maximum((z - mu) * inv * g_ref[...] + be_ref[...], 0.0)
    h3 = jnp.maximum(
        lax.dot_general(h2, wh1_ref[...], (((1,), (1,)), ((), ())),
                        preferred_element_type=jnp.float32) + bh1_ref[...],
        0.0)
    out_ref[...] = (lax.dot_general(h3, wh2_ref[...], (((1,), (1,)), ((), ())),
                                    preferred_element_type=jnp.float32)
                    + bh2_ref[0, 0])


def _tc_d(h2pre, stats2, g2, be2, Wh1, bh1, Wh2, bh2):
    grid = (N // RB,)
    return pl.pallas_call(
        _tc_d_body,
        grid=grid,
        in_specs=[
            pl.BlockSpec((RB, 32), lambda i: (i, 0)),
            pl.BlockSpec((2, 32), lambda i: (0, 0)),
            pl.BlockSpec((1, 32), lambda i: (0, 0)),
            pl.BlockSpec((1, 32), lambda i: (0, 0)),
            pl.BlockSpec((16, 32), lambda i: (0, 0)),
            pl.BlockSpec((1, 16), lambda i: (0, 0)),
            pl.BlockSpec((8, 16), lambda i: (0, 0)),
            pl.BlockSpec(memory_space=pltpu.SMEM),
        ],
        out_specs=pl.BlockSpec((RB, 8), lambda i: (i, 0)),
        out_shape=jax.ShapeDtypeStruct((N, 8), jnp.float32),
    )(h2pre, stats2, g2, be2, Wh1, bh1, Wh2, bh2)


def kernel(xs, xt, edge_index, Wl1, bl1, Wr1, g1, be1, Wl2, bl2, Wr2, g2, be2,
           Wh1, bh1, Wh2, bh2):
    x = jnp.concatenate([xs, xt], axis=-1)
    pad_idx = N + jnp.arange(EPAD - E, dtype=jnp.int32) % (NPAD - N)
    edges = jnp.concatenate(
        [edge_index.astype(jnp.int32),
         jnp.broadcast_to(pad_idx, (2, EPAD - E))], axis=1
    ).reshape(2, EROWS, RL)
    zeros = jnp.zeros((NPAD, 16), jnp.float32)

    t1b = jnp.concatenate(
        [x[:, 16:20], jnp.ones((N, 1), jnp.float32),
         jnp.zeros((N, 11), jnp.float32)], axis=1)
    tables1 = jnp.pad(jnp.stack([x[:, :16], t1b]),
                      ((0, 0), (0, NPAD - N), (0, 0)))

    sums1 = _sc_segsum(tables1, edges, zeros)

    h1pre, stats1, cnt = _tc_a(sums1, x, Wl1, bl1.reshape(1, 64), Wr1)
    p2, r2 = _tc_b(h1pre, stats1, g1.reshape(1, 64), be1.reshape(1, 64),
                   Wl2, bl2.reshape(1, 32), Wr2)

    sums2 = _sc_segsum(p2, edges, zeros)

    h2pre, stats2 = _tc_c(sums2, cnt, r2)
    out = _tc_d(h2pre, stats2, g2.reshape(1, 32), be2.reshape(1, 32),
                Wh1, bh1.reshape(1, 16),
                jnp.pad(Wh2, ((0, 7), (0, 0))), bh2.reshape(1, 1))
    return out[:, 0]


# speedup vs baseline: 1.5511x; 1.0000x over previous
import functools

import jax
import jax.numpy as jnp
from jax import lax
from jax.experimental import pallas as pl
from jax.experimental.pallas import tpu as pltpu
from jax.experimental.pallas import tpu_sc as plsc

N = 100000
NPAD = 102400
E = 3200000
RL = 128
EROWS = 25728
EPAD = EROWS * RL
K = 6
NSUB = 16
NPS = NPAD // NSUB
RB = 2000
EPS = 1e-5


def _sc_segsum(tables, edges, zeros):
    chunks_total = EROWS // K
    chunks_per_tile = chunks_total // NSUB
    pairs = chunks_per_tile // 2

    mesh = plsc.VectorSubcoreMesh(core_axis_name="c", subcore_axis_name="s")

    @functools.partial(
        pl.kernel,
        mesh=mesh,
        out_type=jax.ShapeDtypeStruct((2, NPAD, 16), jnp.float32),
        scratch_types=[
            pltpu.VMEM((2, K, RL), jnp.int32),
            pltpu.VMEM((2, K, RL), jnp.int32),
            pltpu.VMEM((2, K, RL, 16), jnp.float32),
            pltpu.VMEM_SHARED((NPAD, 16), jnp.float32),
            pltpu.SemaphoreType.DMA,
            pltpu.SemaphoreType.DMA,
            pltpu.SemaphoreType.DMA,
            pltpu.SemaphoreType.DMA,
        ],
        compiler_params=pltpu.CompilerParams(use_tc_tiling_on_sc=False),
    )
    def segsum(tables_hbm, edges_hbm, zeros_hbm, out_hbm,
               sidx, didx, rows, acc, gsem0, gsem1, ssem0, ssem1):
        cid = lax.axis_index("c")
        sid = lax.axis_index("s")
        gsem = (gsem0, gsem1)
        ssem = (ssem0, ssem1)
        pltpu.sync_copy(zeros_hbm.at[pl.ds(sid * NPS, NPS)],
                        acc.at[pl.ds(sid * NPS, NPS)])
        plsc.subcore_barrier()

        base = sid * chunks_per_tile

        HL = RL // 2

        def load_and_gather(cb, b):
            pltpu.sync_copy(edges_hbm.at[0].at[pl.ds(cb * K, K)], sidx.at[b])
            pltpu.sync_copy(edges_hbm.at[1].at[pl.ds(cb * K, K)], didx.at[b])
            for j in range(K):
                for h in range(2):
                    pltpu.async_copy(
                        tables_hbm.at[cid].at[sidx.at[b].at[j, pl.ds(h * HL, HL)]],
                        rows.at[b].at[j].at[pl.ds(h * HL, HL)], gsem[b])

        def wait_gathers(b):
            for j in range(K):
                for h in range(2):
                    pltpu.make_async_copy(
                        tables_hbm.at[cid].at[sidx.at[b].at[j, pl.ds(h * HL, HL)]],
                        rows.at[b].at[j].at[pl.ds(h * HL, HL)], gsem[b]).wait()

        def scatter_and_drain(b):
            cps = [
                pltpu.async_copy(
                    rows.at[b].at[j].at[pl.ds(h * HL, HL)],
                    acc.at[didx.at[b].at[j, pl.ds(h * HL, HL)]],
                    ssem[b], add=True)
                for j in range(K) for h in range(2)
            ]
            for cp in cps:
                cp.wait()

        for b in range(2):
            load_and_gather(base + b, b)

        def pair(o, carry):
            for b in range(2):
                cb = base + o * 2 + b
                wait_gathers(b)
                scatter_and_drain(b)

                @pl.when(o < pairs - 1)
                def _():
                    load_and_gather(cb + 2, b)
            return carry

        lax.fori_loop(0, pairs, pair, 0)
        plsc.subcore_barrier()
        pltpu.sync_copy(acc.at[pl.ds(sid * NPS, NPS)],
                        out_hbm.at[cid].at[pl.ds(sid * NPS, NPS)])

    return segsum(tables, edges, zeros)


def _tc_a_body(sums_ref, x_ref, wl_ref, bl_ref, wr_ref,
               h1pre_ref, stats_ref, cnt8_ref, ssum, ssq):
    i = pl.program_id(0)
    s = sums_ref[...]
    cnt8_ref[...] = s[1][:, :8]
    agg = jnp.concatenate([s[0], s[1][:, :4]], axis=1)
    cnt = jnp.maximum(s[1][:, 4:5], 1.0)
    agg = agg / cnt
    z = (lax.dot_general(agg, wl_ref[...], (((1,), (1,)), ((), ())),
                         preferred_element_type=jnp.float32)
         + bl_ref[...]
         + lax.dot_general(x_ref[...], wr_ref[...], (((1,), (1,)), ((), ())),
                           preferred_element_type=jnp.float32))
    h1pre_ref[...] = z

    @pl.when(i == 0)
    def _():
        ssum[...] = jnp.zeros_like(ssum)
        ssq[...] = jnp.zeros_like(ssq)

    ssum[...] += jnp.sum(z, axis=0, keepdims=True)
    ssq[...] += jnp.sum(z * z, axis=0, keepdims=True)

    @pl.when(i == pl.num_programs(0) - 1)
    def _():
        stats_ref[...] = jnp.concatenate([ssum[...], ssq[...]], axis=0)


def _tc_a(sums1, x, Wl1, bl1, Wr1):
    grid = (N // RB,)
    return pl.pallas_call(
        _tc_a_body,
        grid=grid,
        in_specs=[
            pl.BlockSpec((2, RB, 16), lambda i: (0, i, 0)),
            pl.BlockSpec((RB, 20), lambda i: (i, 0)),
            pl.BlockSpec((64, 20), lambda i: (0, 0)),
            pl.BlockSpec((1, 64), lambda i: (0, 0)),
            pl.BlockSpec((64, 20), lambda i: (0, 0)),
        ],
        out_specs=[
            pl.BlockSpec((RB, 64), lambda i: (i, 0)),
            pl.BlockSpec((2, 64), lambda i: (0, 0)),
            pl.BlockSpec((RB, 8), lambda i: (i, 0)),
        ],
        out_shape=[
            jax.ShapeDtypeStruct((N, 64), jnp.float32),
            jax.ShapeDtypeStruct((2, 64), jnp.float32),
            jax.ShapeDtypeStruct((N, 8), jnp.float32),
        ],
        scratch_shapes=[
            pltpu.VMEM((1, 64), jnp.float32),
            pltpu.VMEM((1, 64), jnp.float32),
        ],
    )(sums1, x, Wl1, bl1, Wr1)


def _tc_b_body(h1pre_ref, stats_ref, g_ref, be_ref, wl2_ref, bl2_ref, wr2_ref,
               p2_ref, r2_ref):
    st = stats_ref[...]
    mu = st[0:1, :] / N
    var = st[1:2, :] / N - mu * mu
    inv = lax.rsqrt(var + EPS)
    z = h1pre_ref[...]
    h1 = jnp.maximum((z - mu) * inv * g_ref[...] + be_ref[...], 0.0)
    p2 = lax.dot_general(h1, wl2_ref[...], (((1,), (1,)), ((), ())),
                         preferred_element_type=jnp.float32)
    p2_ref[0] = p2[:, :16]
    p2_ref[1] = p2[:, 16:]
    r2_ref[...] = (lax.dot_general(h1, wr2_ref[...], (((1,), (1,)), ((), ())),
                                   preferred_element_type=jnp.float32)
                   + bl2_ref[...])


def _tc_b(h1pre, stats1, g1, be1, Wl2, bl2, Wr2):
    grid = (N // RB,)
    return pl.pallas_call(
        _tc_b_body,
        grid=grid,
        in_specs=[
            pl.BlockSpec((RB, 64), lambda i: (i, 0)),
            pl.BlockSpec((2, 64), lambda i: (0, 0)),
            pl.BlockSpec((1, 64), lambda i: (0, 0)),
            pl.BlockSpec((1, 64), lambda i: (0, 0)),
            pl.BlockSpec((32, 64), lambda i: (0, 0)),
            pl.BlockSpec((1, 32), lambda i: (0, 0)),
            pl.BlockSpec((32, 64), lambda i: (0, 0)),
        ],
        out_specs=[
            pl.BlockSpec((2, RB, 16), lambda i: (0, i, 0)),
            pl.BlockSpec((RB, 32), lambda i: (i, 0)),
        ],
        out_shape=[
            jax.ShapeDtypeStruct((2, NPAD, 16), jnp.float32),
            jax.ShapeDtypeStruct((N, 32), jnp.float32),
        ],
    )(h1pre, stats1, g1, be1, Wl2, bl2, Wr2)


def _tc_c_body(sums_ref, cnt_ref, r2_ref, h2pre_ref, stats_ref, ssum, ssq):
    i = pl.program_id(0)
    s = sums_ref[...]
    agg = jnp.concatenate([s[0], s[1]], axis=1)
    cnt = jnp.maximum(cnt_ref[:, 4:5], 1.0)
    z = agg / cnt + r2_ref[...]
    h2pre_ref[...] = z

    @pl.when(i == 0)
    def _():
        ssum[...] = jnp.zeros_like(ssum)
        ssq[...] = jnp.zeros_like(ssq)

    ssum[...] += jnp.sum(z, axis=0, keepdims=True)
    ssq[...] += jnp.sum(z * z, axis=0, keepdims=True)

    @pl.when(i == pl.num_programs(0) - 1)
    def _():
        stats_ref[...] = jnp.concatenate([ssum[...], ssq[...]], axis=0)


def _tc_c(sums2, cnt, r2):
    grid = (N // RB,)
    return pl.pallas_call(
        _tc_c_body,
        grid=grid,
        in_specs=[
            pl.BlockSpec((2, RB, 16), lambda i: (0, i, 0)),
            pl.BlockSpec((RB, 8), lambda i: (i, 0)),
            pl.BlockSpec((RB, 32), lambda i: (i, 0)),
        ],
        out_specs=[
            pl.BlockSpec((RB, 32), lambda i: (i, 0)),
            pl.BlockSpec((2, 32), lambda i: (0, 0)),
        ],
        out_shape=[
            jax.ShapeDtypeStruct((N, 32), jnp.float32),
            jax.ShapeDtypeStruct((2, 32), jnp.float32),
        ],
        scratch_shapes=[
            pltpu.VMEM((1, 32), jnp.float32),
            pltpu.VMEM((1, 32), jnp.float32),
        ],
    )(sums2, cnt, r2)


def _tc_d_body(h2pre_ref, stats_ref, g_ref, be_ref, wh1_ref, bh1_ref,
               wh2_ref, bh2_ref, out_ref):
    st = stats_ref[...]
    mu = st[0:1, :] / N
    var = st[1:2, :] / N - mu * mu
    inv = lax.rsqrt(var + EPS)
    z = h2pre_ref[...]
    h2 = jnp.maximum((z - mu) * inv * g_ref[...] + be_ref[...], 0.0)
    h3 = jnp.maximum(
        lax.dot_general(h2, wh1_ref[...], (((1,), (1,)), ((), ())),
                        preferred_element_type=jnp.float32) + bh1_ref[...],
        0.0)
    out_ref[...] = (lax.dot_general(h3, wh2_ref[...], (((1,), (1,)), ((), ())),
                                    preferred_element_type=jnp.float32)
                    + bh2_ref[0, 0])


def _tc_d(h2pre, stats2, g2, be2, Wh1, bh1, Wh2, bh2):
    grid = (N // RB,)
    return pl.pallas_call(
        _tc_d_body,
        grid=grid,
        in_specs=[
            pl.BlockSpec((RB, 32), lambda i: (i, 0)),
            pl.BlockSpec((2, 32), lambda i: (0, 0)),
            pl.BlockSpec((1, 32), lambda i: (0, 0)),
            pl.BlockSpec((1, 32), lambda i: (0, 0)),
            pl.BlockSpec((16, 32), lambda i: (0, 0)),
            pl.BlockSpec((1, 16), lambda i: (0, 0)),
            pl.BlockSpec((8, 16), lambda i: (0, 0)),
            pl.BlockSpec(memory_space=pltpu.SMEM),
        ],
        out_specs=pl.BlockSpec((RB, 8), lambda i: (i, 0)),
        out_shape=jax.ShapeDtypeStruct((N, 8), jnp.float32),
    )(h2pre, stats2, g2, be2, Wh1, bh1, Wh2, bh2)


def kernel(xs, xt, edge_index, Wl1, bl1, Wr1, g1, be1, Wl2, bl2, Wr2, g2, be2,
           Wh1, bh1, Wh2, bh2):
    x = jnp.concatenate([xs, xt], axis=-1)
    pad_idx = N + jnp.arange(EPAD - E, dtype=jnp.int32) % (NPAD - N)
    edges = jnp.concatenate(
        [edge_index.astype(jnp.int32),
         jnp.broadcast_to(pad_idx, (2, EPAD - E))], axis=1
    ).reshape(2, EROWS, RL)
    zeros = jnp.zeros((NPAD, 16), jnp.float32)

    t1b = jnp.concatenate(
        [x[:, 16:20], jnp.ones((N, 1), jnp.float32),
         jnp.zeros((N, 11), jnp.float32)], axis=1)
    tables1 = jnp.pad(jnp.stack([x[:, :16], t1b]),
                      ((0, 0), (0, NPAD - N), (0, 0)))

    sums1 = _sc_segsum(tables1, edges, zeros)

    h1pre, stats1, cnt = _tc_a(sums1, x, Wl1, bl1.reshape(1, 64), Wr1)
    p2, r2 = _tc_b(h1pre, stats1, g1.reshape(1, 64), be1.reshape(1, 64),
                   Wl2, bl2.reshape(1, 32), Wr2)

    sums2 = _sc_segsum(p2, edges, zeros)

    h2pre, stats2 = _tc_c(sums2, cnt, r2)
    out = _tc_d(h2pre, stats2, g2.reshape(1, 32), be2.reshape(1, 32),
                Wh1, bh1.reshape(1, 16),
                jnp.pad(Wh2, ((0, 7), (0, 0))), bh2.reshape(1, 1))
    return out[:, 0]


# speedup vs baseline: 1.5930x; 1.0270x over previous
import functools

import jax
import jax.numpy as jnp
from jax import lax
from jax.experimental import pallas as pl
from jax.experimental.pallas import tpu as pltpu
from jax.experimental.pallas import tpu_sc as plsc

N = 100000
NPAD = 102400
E = 3200000
RL = 128
EROWS = 25728
EPAD = EROWS * RL
K = 6
NSUB = 16
NPS = NPAD // NSUB
RB = 4000
EPS = 1e-5


def _sc_segsum(tables, edges, zeros):
    chunks_total = EROWS // K
    chunks_per_tile = chunks_total // NSUB
    pairs = chunks_per_tile // 2

    mesh = plsc.VectorSubcoreMesh(core_axis_name="c", subcore_axis_name="s")

    @functools.partial(
        pl.kernel,
        mesh=mesh,
        out_type=jax.ShapeDtypeStruct((2, NPAD, 16), jnp.float32),
        scratch_types=[
            pltpu.VMEM((2, K, RL), jnp.int32),
            pltpu.VMEM((2, K, RL), jnp.int32),
            pltpu.VMEM((2, K, RL, 16), jnp.float32),
            pltpu.VMEM_SHARED((NPAD, 16), jnp.float32),
            pltpu.SemaphoreType.DMA,
            pltpu.SemaphoreType.DMA,
            pltpu.SemaphoreType.DMA,
            pltpu.SemaphoreType.DMA,
        ],
        compiler_params=pltpu.CompilerParams(use_tc_tiling_on_sc=False),
    )
    def segsum(tables_hbm, edges_hbm, zeros_hbm, out_hbm,
               sidx, didx, rows, acc, gsem0, gsem1, ssem0, ssem1):
        cid = lax.axis_index("c")
        sid = lax.axis_index("s")
        gsem = (gsem0, gsem1)
        ssem = (ssem0, ssem1)
        pltpu.sync_copy(zeros_hbm.at[pl.ds(sid * NPS, NPS)],
                        acc.at[pl.ds(sid * NPS, NPS)])
        plsc.subcore_barrier()

        base = sid * chunks_per_tile

        HL = RL // 2

        def load_and_gather(cb, b):
            pltpu.sync_copy(edges_hbm.at[0].at[pl.ds(cb * K, K)], sidx.at[b])
            pltpu.sync_copy(edges_hbm.at[1].at[pl.ds(cb * K, K)], didx.at[b])
            for j in range(K):
                for h in range(2):
                    pltpu.async_copy(
                        tables_hbm.at[cid].at[sidx.at[b].at[j, pl.ds(h * HL, HL)]],
                        rows.at[b].at[j].at[pl.ds(h * HL, HL)], gsem[b])

        def wait_gathers(b):
            for j in range(K):
                for h in range(2):
                    pltpu.make_async_copy(
                        tables_hbm.at[cid].at[sidx.at[b].at[j, pl.ds(h * HL, HL)]],
                        rows.at[b].at[j].at[pl.ds(h * HL, HL)], gsem[b]).wait()

        def scatter_and_drain(b):
            cps = [
                pltpu.async_copy(
                    rows.at[b].at[j].at[pl.ds(h * HL, HL)],
                    acc.at[didx.at[b].at[j, pl.ds(h * HL, HL)]],
                    ssem[b], add=True)
                for j in range(K) for h in range(2)
            ]
            for cp in cps:
                cp.wait()

        for b in range(2):
            load_and_gather(base + b, b)

        def pair(o, carry):
            for b in range(2):
                cb = base + o * 2 + b
                wait_gathers(b)
                scatter_and_drain(b)

                @pl.when(o < pairs - 1)
                def _():
                    load_and_gather(cb + 2, b)
            return carry

        lax.fori_loop(0, pairs, pair, 0)
        plsc.subcore_barrier()
        pltpu.sync_copy(acc.at[pl.ds(sid * NPS, NPS)],
                        out_hbm.at[cid].at[pl.ds(sid * NPS, NPS)])

    return segsum(tables, edges, zeros)


def _tc_a_body(sums_ref, x_ref, wl_ref, bl_ref, wr_ref,
               h1pre_ref, stats_ref, cnt8_ref, ssum, ssq):
    i = pl.program_id(0)
    s = sums_ref[...]
    cnt8_ref[...] = s[1][:, :8]
    agg = jnp.concatenate([s[0], s[1][:, :4]], axis=1)
    cnt = jnp.maximum(s[1][:, 4:5], 1.0)
    agg = agg / cnt
    z = (lax.dot_general(agg, wl_ref[...], (((1,), (1,)), ((), ())),
                         preferred_element_type=jnp.float32)
         + bl_ref[...]
         + lax.dot_general(x_ref[...], wr_ref[...], (((1,), (1,)), ((), ())),
                           preferred_element_type=jnp.float32))
    h1pre_ref[...] = z

    @pl.when(i == 0)
    def _():
        ssum[...] = jnp.zeros_like(ssum)
        ssq[...] = jnp.zeros_like(ssq)

    ssum[...] += jnp.sum(z, axis=0, keepdims=True)
    ssq[...] += jnp.sum(z * z, axis=0, keepdims=True)

    @pl.when(i == pl.num_programs(0) - 1)
    def _():
        stats_ref[...] = jnp.concatenate([ssum[...], ssq[...]], axis=0)


def _tc_a(sums1, x, Wl1, bl1, Wr1):
    grid = (N // RB,)
    return pl.pallas_call(
        _tc_a_body,
        grid=grid,
        in_specs=[
            pl.BlockSpec((2, RB, 16), lambda i: (0, i, 0)),
            pl.BlockSpec((RB, 20), lambda i: (i, 0)),
            pl.BlockSpec((64, 20), lambda i: (0, 0)),
            pl.BlockSpec((1, 64), lambda i: (0, 0)),
            pl.BlockSpec((64, 20), lambda i: (0, 0)),
        ],
        out_specs=[
            pl.BlockSpec((RB, 64), lambda i: (i, 0)),
            pl.BlockSpec((2, 64), lambda i: (0, 0)),
            pl.BlockSpec((RB, 8), lambda i: (i, 0)),
        ],
        out_shape=[
            jax.ShapeDtypeStruct((N, 64), jnp.float32),
            jax.ShapeDtypeStruct((2, 64), jnp.float32),
            jax.ShapeDtypeStruct((N, 8), jnp.float32),
        ],
        scratch_shapes=[
            pltpu.VMEM((1, 64), jnp.float32),
            pltpu.VMEM((1, 64), jnp.float32),
        ],
    )(sums1, x, Wl1, bl1, Wr1)


def _tc_b_body(h1pre_ref, stats_ref, g_ref, be_ref, wl2_ref, bl2_ref, wr2_ref,
               p2_ref, r2_ref):
    st = stats_ref[...]
    mu = st[0:1, :] / N
    var = st[1:2, :] / N - mu * mu
    inv = lax.rsqrt(var + EPS)
    z = h1pre_ref[...]
    h1 = jnp.maximum((z - mu) * inv * g_ref[...] + be_ref[...], 0.0)
    p2 = lax.dot_general(h1, wl2_ref[...], (((1,), (1,)), ((), ())),
                         preferred_element_type=jnp.float32)
    p2_ref[0] = p2[:, :16]
    p2_ref[1] = p2[:, 16:]
    r2_ref[...] = (lax.dot_general(h1, wr2_ref[...], (((1,), (1,)), ((), ())),
                                   preferred_element_type=jnp.float32)
                   + bl2_ref[...])


def _tc_b(h1pre, stats1, g1, be1, Wl2, bl2, Wr2):
    grid = (N // RB,)
    return pl.pallas_call(
        _tc_b_body,
        grid=grid,
        in_specs=[
            pl.BlockSpec((RB, 64), lambda i: (i, 0)),
            pl.BlockSpec((2, 64), lambda i: (0, 0)),
            pl.BlockSpec((1, 64), lambda i: (0, 0)),
            pl.BlockSpec((1, 64), lambda i: (0, 0)),
            pl.BlockSpec((32, 64), lambda i: (0, 0)),
            pl.BlockSpec((1, 32), lambda i: (0, 0)),
            pl.BlockSpec((32, 64), lambda i: (0, 0)),
        ],
        out_specs=[
            pl.BlockSpec((2, RB, 16), lambda i: (0, i, 0)),
            pl.BlockSpec((RB, 32), lambda i: (i, 0)),
        ],
        out_shape=[
            jax.ShapeDtypeStruct((2, NPAD, 16), jnp.float32),
            jax.ShapeDtypeStruct((N, 32), jnp.float32),
        ],
    )(h1pre, stats1, g1, be1, Wl2, bl2, Wr2)


def _tc_c_body(sums_ref, cnt_ref, r2_ref, h2pre_ref, stats_ref, ssum, ssq):
    i = pl.program_id(0)
    s = sums_ref[...]
    agg = jnp.concatenate([s[0], s[1]], axis=1)
    cnt = jnp.maximum(cnt_ref[:, 4:5], 1.0)
    z = agg / cnt + r2_ref[...]
    h2pre_ref[...] = z

    @pl.when(i == 0)
    def _():
        ssum[...] = jnp.zeros_like(ssum)
        ssq[...] = jnp.zeros_like(ssq)

    ssum[...] += jnp.sum(z, axis=0, keepdims=True)
    ssq[...] += jnp.sum(z * z, axis=0, keepdims=True)

    @pl.when(i == pl.num_programs(0) - 1)
    def _():
        stats_ref[...] = jnp.concatenate([ssum[...], ssq[...]], axis=0)


def _tc_c(sums2, cnt, r2):
    grid = (N // RB,)
    return pl.pallas_call(
        _tc_c_body,
        grid=grid,
        in_specs=[
            pl.BlockSpec((2, RB, 16), lambda i: (0, i, 0)),
            pl.BlockSpec((RB, 8), lambda i: (i, 0)),
            pl.BlockSpec((RB, 32), lambda i: (i, 0)),
        ],
        out_specs=[
            pl.BlockSpec((RB, 32), lambda i: (i, 0)),
            pl.BlockSpec((2, 32), lambda i: (0, 0)),
        ],
        out_shape=[
            jax.ShapeDtypeStruct((N, 32), jnp.float32),
            jax.ShapeDtypeStruct((2, 32), jnp.float32),
        ],
        scratch_shapes=[
            pltpu.VMEM((1, 32), jnp.float32),
            pltpu.VMEM((1, 32), jnp.float32),
        ],
    )(sums2, cnt, r2)


def _tc_d_body(h2pre_ref, stats_ref, g_ref, be_ref, wh1_ref, bh1_ref,
               wh2_ref, bh2_ref, out_ref):
    st = stats_ref[...]
    mu = st[0:1, :] / N
    var = st[1:2, :] / N - mu * mu
    inv = lax.rsqrt(var + EPS)
    z = h2pre_ref[...]
    h2 = jnp.maximum((z - mu) * inv * g_ref[...] + be_ref[...], 0.0)
    h3 = jnp.maximum(
        lax.dot_general(h2, wh1_ref[...], (((1,), (1,)), ((), ())),
                        preferred_element_type=jnp.float32) + bh1_ref[...],
        0.0)
    out_ref[...] = (lax.dot_general(h3, wh2_ref[...], (((1,), (1,)), ((), ())),
                                    preferred_element_type=jnp.float32)
                    + bh2_ref[0, 0])


def _tc_d(h2pre, stats2, g2, be2, Wh1, bh1, Wh2, bh2):
    grid = (N // RB,)
    return pl.pallas_call(
        _tc_d_body,
        grid=grid,
        in_specs=[
            pl.BlockSpec((RB, 32), lambda i: (i, 0)),
            pl.BlockSpec((2, 32), lambda i: (0, 0)),
            pl.BlockSpec((1, 32), lambda i: (0, 0)),
            pl.BlockSpec((1, 32), lambda i: (0, 0)),
            pl.BlockSpec((16, 32), lambda i: (0, 0)),
            pl.BlockSpec((1, 16), lambda i: (0, 0)),
            pl.BlockSpec((8, 16), lambda i: (0, 0)),
            pl.BlockSpec(memory_space=pltpu.SMEM),
        ],
        out_specs=pl.BlockSpec((RB, 8), lambda i: (i, 0)),
        out_shape=jax.ShapeDtypeStruct((N, 8), jnp.float32),
    )(h2pre, stats2, g2, be2, Wh1, bh1, Wh2, bh2)


def kernel(xs, xt, edge_index, Wl1, bl1, Wr1, g1, be1, Wl2, bl2, Wr2, g2, be2,
           Wh1, bh1, Wh2, bh2):
    x = jnp.concatenate([xs, xt], axis=-1)
    pad_idx = N + jnp.arange(EPAD - E, dtype=jnp.int32) % (NPAD - N)
    edges = jnp.concatenate(
        [edge_index.astype(jnp.int32),
         jnp.broadcast_to(pad_idx, (2, EPAD - E))], axis=1
    ).reshape(2, EROWS, RL)
    zeros = jnp.zeros((NPAD, 16), jnp.float32)

    t1b = jnp.concatenate(
        [x[:, 16:20], jnp.ones((N, 1), jnp.float32),
         jnp.zeros((N, 11), jnp.float32)], axis=1)
    tables1 = jnp.pad(jnp.stack([x[:, :16], t1b]),
                      ((0, 0), (0, NPAD - N), (0, 0)))

    sums1 = _sc_segsum(tables1, edges, zeros)

    h1pre, stats1, cnt = _tc_a(sums1, x, Wl1, bl1.reshape(1, 64), Wr1)
    p2, r2 = _tc_b(h1pre, stats1, g1.reshape(1, 64), be1.reshape(1, 64),
                   Wl2, bl2.reshape(1, 32), Wr2)

    sums2 = _sc_segsum(p2, edges, zeros)

    h2pre, stats2 = _tc_c(sums2, cnt, r2)
    out = _tc_d(h2pre, stats2, g2.reshape(1, 32), be2.reshape(1, 32),
                Wh1, bh1.reshape(1, 16),
                jnp.pad(Wh2, ((0, 7), (0, 0))), bh2.reshape(1, 1))
    return out[:, 0]
